# baseline (speedup 1.0000x reference)
import jax
import jax.numpy as jnp
from jax import lax
from jax.experimental import pallas as pl
from jax.experimental.pallas import tpu as pltpu
from jax.experimental.pallas import tpu_sc as plsc

B = 1024
L = 50
G = 10
D = 64
NC = 2
NS = 16
NW = NC * NS
BPW = B // NW
TPW = BPW * L
NIDX = TPW * G
IDX_PER_B = L * G
IDX_PAD = 512
LANES = 16

NWN = 1000001
NWL = 1000002
TCOLS = 128
NBLK = (NWL + TCOLS - 1) // TCOLS
NWL_PAD = NBLK * TCOLS


def _tr_body(in_ref, out_ref, scr):
    scr[:, 0:D] = jnp.transpose(in_ref[...], (1, 0))
    out_ref[:, 0:D] = scr[pl.Slice(0, TCOLS // 2, 2), 0:D]
    out_ref[:, D:2 * D] = scr[pl.Slice(1, TCOLS // 2, 2), 0:D]


def _relayout_wl(wlT):
    return pl.pallas_call(
        _tr_body,
        grid=(NBLK,),
        in_specs=[pl.BlockSpec((D, TCOLS), lambda j: (0, j))],
        out_specs=pl.BlockSpec((TCOLS // 2, 128), lambda j: (j, 0)),
        out_shape=jax.ShapeDtypeStruct((NWL_PAD // 2, 128), jnp.float32),
        scratch_shapes=[pltpu.VMEM((TCOLS, 128), jnp.float32)],
    )(wlT)


def _body(sent_ref, wn_ref, wl_ref, out_ref,
          sent_v, idx1_v, ng_v, idx_v, rows_v, out_v, sem0, sems):
    wid = lax.axis_index("s") * NC + lax.axis_index("c")

    pltpu.sync_copy(sent_ref.at[pl.ds(wid * TPW, TPW)], sent_v)

    iota = lax.iota(jnp.int32, LANES)

    def idx1_body(k, _):
        s = k * LANES + iota
        t = lax.shift_right_logical(s * 6554, 16)
        g = s - t * G
        tok = plsc.load_gather(sent_v, [t])
        idx1_v[k // 8, pl.ds((k % 8) * LANES, LANES)] = g * NWN + tok
        return 0

    lax.fori_loop(0, NIDX // LANES, idx1_body, 0)

    hop1 = [
        pltpu.async_copy(wn_ref.at[idx1_v.at[j]], ng_v.at[j], sem0)
        for j in range(NIDX // 128)
    ]
    for cp in hop1:
        cp.wait()

    def conv_body(t, _):
        b = t // 32
        k = t - b * 32
        off = k * LANES + iota
        valid = off < IDX_PER_B
        p = b * IDX_PER_B + off
        r = jnp.minimum(lax.shift_right_logical(p, 7), NIDX // 128 - 1)
        c = lax.bitwise_and(p, 127)
        v = plsc.load_gather(ng_v, [r, c])
        vi = jnp.where(valid, v.astype(jnp.int32), 0)
        idx_v[b, k // 8, pl.ds((k % 8) * LANES, LANES)] = vi
        return 0

    lax.fori_loop(0, BPW * 32, conv_body, 0)

    def fire(b, par):
        return [
            pltpu.async_copy(wl_ref.at[idx_v.at[b, j]],
                             rows_v.at[par, pl.ds(j * 128, 128)],
                             sems.at[par])
            for j in range(4)
        ]

    pending = {0: fire(0, 0)}
    for b in range(BPW):
        par = b % 2
        if b + 1 < BPW:
            pending[1 - par] = fire(b + 1, 1 - par)
        for cp in pending[par]:
            cp.wait()

        def red_body(rr, accs):
            a0, a1, a2, a3 = accs
            for u in range(4):
                r = rr * 4 + u
                a0 = a0 + rows_v[par, r, pl.ds(0, LANES)]
                a1 = a1 + rows_v[par, r, pl.ds(LANES, LANES)]
                a2 = a2 + rows_v[par, r, pl.ds(2 * LANES, LANES)]
                a3 = a3 + rows_v[par, r, pl.ds(3 * LANES, LANES)]
            return a0, a1, a2, a3

        z = jnp.zeros((LANES,), jnp.float32)
        acc = lax.fori_loop(0, IDX_PAD // 4, red_body, (z, z, z, z))
        for d in range(4):
            out_v[b, pl.ds(d * LANES, LANES)] = acc[d]

    pltpu.sync_copy(out_v, out_ref.at[pl.ds(wid * BPW, BPW)])


@jax.jit
def _run(sent_f, wn_f, wl2):
    mesh = plsc.VectorSubcoreMesh(core_axis_name="c", subcore_axis_name="s")
    return pl.kernel(
        _body,
        out_type=jax.ShapeDtypeStruct((B, D), jnp.float32),
        mesh=mesh,
        scratch_types=[
            pltpu.VMEM((TPW,), jnp.int32),
            pltpu.VMEM((NIDX // 128, 128), jnp.int32),
            pltpu.VMEM((NIDX // 128, 128), jnp.float32),
            pltpu.VMEM((BPW, 4, 128), jnp.int32),
            pltpu.VMEM((2, IDX_PAD, D), jnp.float32),
            pltpu.VMEM((BPW, D), jnp.float32),
            pltpu.SemaphoreType.DMA,
            pltpu.SemaphoreType.DMA((2,)),
        ],
        compiler_params=pltpu.CompilerParams(use_tc_tiling_on_sc=False,
                                             needs_layout_passes=False),
    )(sent_f, wn_f, wl2)


def kernel(sent, W_ngram_idx, W_lookup):
    sent_f = sent.astype(jnp.int32).reshape(-1)
    wn_f = W_ngram_idx.T.reshape(-1)
    wl_lin = _relayout_wl(W_lookup.T)
    wl2 = wl_lin.reshape(NWL_PAD, D)
    return _run(sent_f, wn_f, wl2)


# speedup vs baseline: 2.8556x; 2.8556x over previous
import jax
import jax.numpy as jnp
from jax import lax
from jax.experimental import pallas as pl
from jax.experimental.pallas import tpu as pltpu
from jax.experimental.pallas import tpu_sc as plsc

B = 1024
L = 50
G = 10
D = 64
NC = 2
NS = 16
NW = NC * NS
BPW = B // NW
TPW = BPW * L
TPAD = 1664
IDX_PER_B = L * G
IDX_PAD = 512
LANES = 16

NWN = 1000001
NWL = 1000002
TCOLS = 1024
NBLK = (NWL + TCOLS - 1) // TCOLS
NWL_PAD = NBLK * TCOLS


def _tr_body(in_ref, out_ref, scr):
    scr[:, 0:D] = jnp.transpose(in_ref[...], (1, 0))
    out_ref[:, 0:D] = scr[pl.Slice(0, TCOLS // 2, 2), 0:D]
    out_ref[:, D:2 * D] = scr[pl.Slice(1, TCOLS // 2, 2), 0:D]


def _relayout_wl(wlT):
    return pl.pallas_call(
        _tr_body,
        grid=(NBLK,),
        in_specs=[pl.BlockSpec((D, TCOLS), lambda j: (0, j))],
        out_specs=pl.BlockSpec((TCOLS // 2, 128), lambda j: (j, 0)),
        out_shape=jax.ShapeDtypeStruct((NWL_PAD // 2, 128), jnp.float32),
        scratch_shapes=[pltpu.VMEM((TCOLS, 128), jnp.float32)],
    )(wlT)


def _body(sent_ref, wn_ref, wl_ref, out_ref,
          sent_v, ng_v, idx_v, rows_v, out_v, sem0, sems):
    wid = lax.axis_index("s") * NC + lax.axis_index("c")

    pltpu.sync_copy(sent_ref.at[wid], sent_v)

    hop1 = [
        pltpu.async_copy(wn_ref.at[g].at[sent_v.at[j]],
                         ng_v.at[g * 13 + j], sem0)
        for g in range(G)
        for j in range(13)
    ]
    for cp in hop1:
        cp.wait()

    iota = lax.iota(jnp.int32, LANES)

    def conv_body(tt, _):
        b = tt // 32
        k = tt - b * 32
        off = k * LANES + iota
        valid = off < IDX_PER_B
        p = b * IDX_PER_B + off
        t = lax.shift_right_logical(p * 6554, 16)
        g = p - t * G
        t = jnp.minimum(t, TPW - 1)
        r = g * 13 + lax.shift_right_logical(t, 7)
        c = lax.bitwise_and(t, 127)
        v = plsc.load_gather(ng_v, [r, c])
        vi = jnp.where(valid, v.astype(jnp.int32), 0)
        idx_v[b, k // 8, pl.ds((k % 8) * LANES, LANES)] = vi
        return 0

    lax.fori_loop(0, BPW * 32, conv_body, 0)

    def fire(b, par):
        return [
            pltpu.async_copy(wl_ref.at[idx_v.at[b, j]],
                             rows_v.at[par, pl.ds(j * 128, 128)],
                             sems.at[par])
            for j in range(4)
        ]

    pending = {0: fire(0, 0)}
    for b in range(BPW):
        par = b % 2
        if b + 1 < BPW:
            pending[1 - par] = fire(b + 1, 1 - par)
        for cp in pending[par]:
            cp.wait()

        def red_body(rr, accs):
            a0, a1, a2, a3 = accs
            for u in range(4):
                r = rr * 4 + u
                a0 = a0 + rows_v[par, r, pl.ds(0, LANES)]
                a1 = a1 + rows_v[par, r, pl.ds(LANES, LANES)]
                a2 = a2 + rows_v[par, r, pl.ds(2 * LANES, LANES)]
                a3 = a3 + rows_v[par, r, pl.ds(3 * LANES, LANES)]
            return a0, a1, a2, a3

        z = jnp.zeros((LANES,), jnp.float32)
        acc = lax.fori_loop(0, IDX_PAD // 4, red_body, (z, z, z, z))
        for d in range(4):
            out_v[b, pl.ds(d * LANES, LANES)] = acc[d]

    pltpu.sync_copy(out_v, out_ref.at[pl.ds(wid * BPW, BPW)])


@jax.jit
def _run(sent_p, wnT, wl2):
    mesh = plsc.VectorSubcoreMesh(core_axis_name="c", subcore_axis_name="s")
    return pl.kernel(
        _body,
        out_type=jax.ShapeDtypeStruct((B, D), jnp.float32),
        mesh=mesh,
        scratch_types=[
            pltpu.VMEM((13, 128), jnp.int32),
            pltpu.VMEM((G * 13, 128), jnp.float32),
            pltpu.VMEM((BPW, 4, 128), jnp.int32),
            pltpu.VMEM((2, IDX_PAD, D), jnp.float32),
            pltpu.VMEM((BPW, D), jnp.float32),
            pltpu.SemaphoreType.DMA,
            pltpu.SemaphoreType.DMA((2,)),
        ],
        compiler_params=pltpu.CompilerParams(use_tc_tiling_on_sc=False,
                                             needs_layout_passes=False),
    )(sent_p, wnT, wl2)


def kernel(sent, W_ngram_idx, W_lookup):
    sent_p = jnp.pad(sent.astype(jnp.int32).reshape(NW, TPW),
                     ((0, 0), (0, TPAD - TPW))).reshape(NW, 13, 128)
    wnT = W_ngram_idx.T
    wl_lin = _relayout_wl(W_lookup.T)
    wl2 = wl_lin.reshape(NWL_PAD, D)
    return _run(sent_p, wnT, wl2)


# speedup vs baseline: 5.6057x; 1.9630x over previous
import jax
import jax.numpy as jnp
from jax import lax
from jax.experimental import pallas as pl
from jax.experimental.pallas import tpu as pltpu
from jax.experimental.pallas import tpu_sc as plsc

B = 1024
L = 50
G = 10
D = 64
NC = 2
NS = 16
NW = NC * NS
BPW = B // NW
TPW = BPW * L
NIDX = TPW * G
IDX_PER_B = L * G
IDX_PAD = 512
LANES = 16

NWN = 1000001
NWL = 1000002
TCOLS = 4096
NBLK = (NWL + TCOLS - 1) // TCOLS
NWL_PAD = NBLK * TCOLS

WCH = 2048
WBLK = (NWN + WCH - 1) // WCH
WROWS = WBLK * (WCH // 128) * G
WROWS_PAD = ((WROWS + 7) // 8) * 8


def _tr_body(in_ref, out_ref, scr):
    scr[:, 0:D] = jnp.transpose(in_ref[...], (1, 0))
    out_ref[:, 0:D] = scr[pl.Slice(0, TCOLS // 2, 2), 0:D]
    out_ref[:, D:2 * D] = scr[pl.Slice(1, TCOLS // 2, 2), 0:D]


def _relayout_wl(wlT):
    return pl.pallas_call(
        _tr_body,
        grid=(NBLK,),
        in_specs=[pl.BlockSpec((D, TCOLS), lambda j: (0, j))],
        out_specs=pl.BlockSpec((TCOLS // 2, 128), lambda j: (j, 0)),
        out_shape=jax.ShapeDtypeStruct((NWL_PAD // 2, 128), jnp.float32),
        scratch_shapes=[pltpu.VMEM((TCOLS, 128), jnp.float32)],
    )(wlT)


def _wn_body(in_ref, out_ref):
    for s in range(WCH // 128):
        out_ref[pl.ds(G * s, G), :] = in_ref[:, pl.ds(128 * s, 128)]


def _relayout_wn(wnT):
    return pl.pallas_call(
        _wn_body,
        grid=(WBLK,),
        in_specs=[pl.BlockSpec((G, WCH), lambda j: (0, j))],
        out_specs=pl.BlockSpec((G * WCH // 128, 128), lambda j: (j, 0)),
        out_shape=jax.ShapeDtypeStruct((WROWS_PAD, 128), jnp.float32),
    )(wnT)


def _body(sent_ref, wn_ref, wl_ref, out_ref,
          sent_v, idx1_v, ng_v, idx_v, rows_v, out_v, sem0, sems):
    wid = lax.axis_index("s") * NC + lax.axis_index("c")

    pltpu.sync_copy(sent_ref.at[pl.ds(wid * TPW, TPW)], sent_v)

    iota = lax.iota(jnp.int32, LANES)

    def idx1_body(k, _):
        s = k * LANES + iota
        t = lax.shift_right_logical(s * 6554, 16)
        g = s - t * G
        tok = plsc.load_gather(sent_v, [t])
        e = (G * lax.shift_right_logical(tok, 7) + g) * 128 \
            + lax.bitwise_and(tok, 127)
        idx1_v[k // 8, pl.ds((k % 8) * LANES, LANES)] = e
        return 0

    lax.fori_loop(0, NIDX // LANES, idx1_body, 0)

    hop1 = [
        pltpu.async_copy(wn_ref.at[idx1_v.at[j]], ng_v.at[j], sem0)
        for j in range(NIDX // 128)
    ]
    for cp in hop1:
        cp.wait()

    def conv_body(t, _):
        b = t // 32
        k = t - b * 32
        off = k * LANES + iota
        valid = off < IDX_PER_B
        p = b * IDX_PER_B + off
        r = jnp.minimum(lax.shift_right_logical(p, 7), NIDX // 128 - 1)
        c = lax.bitwise_and(p, 127)
        v = plsc.load_gather(ng_v, [r, c])
        vi = jnp.where(valid, v.astype(jnp.int32), 0)
        idx_v[b, k // 8, pl.ds((k % 8) * LANES, LANES)] = vi
        return 0

    lax.fori_loop(0, BPW * 32, conv_body, 0)

    def fire(b, par):
        return [
            pltpu.async_copy(wl_ref.at[idx_v.at[b, j]],
                             rows_v.at[par, pl.ds(j * 128, 128)],
                             sems.at[par])
            for j in range(4)
        ]

    pending = {0: fire(0, 0)}
    for b in range(BPW):
        par = b % 2
        if b + 1 < BPW:
            pending[1 - par] = fire(b + 1, 1 - par)
        for cp in pending[par]:
            cp.wait()

        def red_body(rr, accs):
            a0, a1, a2, a3 = accs
            for u in range(4):
                r = rr * 4 + u
                a0 = a0 + rows_v[par, r, pl.ds(0, LANES)]
                a1 = a1 + rows_v[par, r, pl.ds(LANES, LANES)]
                a2 = a2 + rows_v[par, r, pl.ds(2 * LANES, LANES)]
                a3 = a3 + rows_v[par, r, pl.ds(3 * LANES, LANES)]
            return a0, a1, a2, a3

        z = jnp.zeros((LANES,), jnp.float32)
        acc = lax.fori_loop(0, IDX_PAD // 4, red_body, (z, z, z, z))
        for d in range(4):
            out_v[b, pl.ds(d * LANES, LANES)] = acc[d]

    pltpu.sync_copy(out_v, out_ref.at[pl.ds(wid * BPW, BPW)])


@jax.jit
def _run(sent_f, wn_f, wl2):
    mesh = plsc.VectorSubcoreMesh(core_axis_name="c", subcore_axis_name="s")
    return pl.kernel(
        _body,
        out_type=jax.ShapeDtypeStruct((B, D), jnp.float32),
        mesh=mesh,
        scratch_types=[
            pltpu.VMEM((TPW,), jnp.int32),
            pltpu.VMEM((NIDX // 128, 128), jnp.int32),
            pltpu.VMEM((NIDX // 128, 128), jnp.float32),
            pltpu.VMEM((BPW, 4, 128), jnp.int32),
            pltpu.VMEM((2, IDX_PAD, D), jnp.float32),
            pltpu.VMEM((BPW, D), jnp.float32),
            pltpu.SemaphoreType.DMA,
            pltpu.SemaphoreType.DMA((2,)),
        ],
        compiler_params=pltpu.CompilerParams(use_tc_tiling_on_sc=False,
                                             needs_layout_passes=False),
    )(sent_f, wn_f, wl2)


def kernel(sent, W_ngram_idx, W_lookup):
    sent_f = sent.astype(jnp.int32).reshape(-1)
    wn_lin = _relayout_wn(W_ngram_idx.T)
    wn_f = wn_lin.reshape(-1)
    wl_lin = _relayout_wl(W_lookup.T)
    wl2 = wl_lin.reshape(NWL_PAD, D)
    return _run(sent_f, wn_f, wl2)


# speedup vs baseline: 7.4122x; 1.3223x over previous
import jax
import jax.numpy as jnp
from jax import lax
from jax.experimental import pallas as pl
from jax.experimental.pallas import tpu as pltpu
from jax.experimental.pallas import tpu_sc as plsc

B = 1024
L = 50
G = 10
D = 64
NC = 2
NS = 16
NW = NC * NS
BPW = B // NW
TPW = BPW * L
NIDX = TPW * G
IDX_PER_B = L * G
IDX_PAD = 512
LANES = 16

NWN = 1000001
NWL = 1000002
TCOLS = 8192
NBLK = (NWL + TCOLS - 1) // TCOLS
NWL_PAD = NBLK * TCOLS

WCH = 8192
WBLK = (NWN + WCH - 1) // WCH
WROWS = WBLK * (WCH // 128) * G
WROWS_PAD = ((WROWS + 7) // 8) * 8


def _tr_body(in_ref, out_ref, scr):
    scr[:, 0:D] = jnp.transpose(in_ref[...], (1, 0))
    out_ref[:, 0:D] = scr[pl.Slice(0, TCOLS // 2, 2), 0:D]
    out_ref[:, D:2 * D] = scr[pl.Slice(1, TCOLS // 2, 2), 0:D]


def _relayout_wl(wlT):
    return pl.pallas_call(
        _tr_body,
        grid=(NBLK,),
        in_specs=[pl.BlockSpec((D, TCOLS), lambda j: (0, j))],
        out_specs=pl.BlockSpec((TCOLS // 2, 128), lambda j: (j, 0)),
        out_shape=jax.ShapeDtypeStruct((NWL_PAD // 2, 128), jnp.float32),
        scratch_shapes=[pltpu.VMEM((TCOLS, 128), jnp.float32)],
    )(wlT)


def _wn_body(in_ref, out_ref):
    for s in range(WCH // 128):
        out_ref[pl.ds(G * s, G), :] = in_ref[:, pl.ds(128 * s, 128)]


def _relayout_wn(wnT):
    return pl.pallas_call(
        _wn_body,
        grid=(WBLK,),
        in_specs=[pl.BlockSpec((G, WCH), lambda j: (0, j))],
        out_specs=pl.BlockSpec((G * WCH // 128, 128), lambda j: (j, 0)),
        out_shape=jax.ShapeDtypeStruct((WROWS_PAD, 128), jnp.float32),
    )(wnT)


def _body(sent_ref, wn_ref, wl_ref, out_ref,
          sent_v, idx1_v, ng_v, idx_v, rows_v, out_v, sem0, sems):
    wid = lax.axis_index("s") * NC + lax.axis_index("c")

    pltpu.sync_copy(sent_ref.at[pl.ds(wid * TPW, TPW)], sent_v)

    iota = lax.iota(jnp.int32, LANES)

    def idx1_body(k, _):
        s = k * LANES + iota
        t = lax.shift_right_logical(s * 6554, 16)
        g = s - t * G
        tok = plsc.load_gather(sent_v, [t])
        e = (G * lax.shift_right_logical(tok, 7) + g) * 128 \
            + lax.bitwise_and(tok, 127)
        idx1_v[k // 8, pl.ds((k % 8) * LANES, LANES)] = e
        return 0

    lax.fori_loop(0, NIDX // LANES, idx1_body, 0)

    hop1 = [
        pltpu.async_copy(wn_ref.at[idx1_v.at[j]], ng_v.at[j], sem0)
        for j in range(NIDX // 128)
    ]
    for cp in hop1:
        cp.wait()

    def conv_body(t, _):
        b = t // 32
        k = t - b * 32
        off = k * LANES + iota
        valid = off < IDX_PER_B
        p = b * IDX_PER_B + off
        r = jnp.minimum(lax.shift_right_logical(p, 7), NIDX // 128 - 1)
        c = lax.bitwise_and(p, 127)
        v = plsc.load_gather(ng_v, [r, c])
        vi = jnp.where(valid, v.astype(jnp.int32), 0)
        idx_v[b, k // 8, pl.ds((k % 8) * LANES, LANES)] = vi
        return 0

    lax.fori_loop(0, BPW * 32, conv_body, 0)

    def fire(b, par):
        return [
            pltpu.async_copy(wl_ref.at[idx_v.at[b, j]],
                             rows_v.at[par, pl.ds(j * 128, 128)],
                             sems.at[par])
            for j in range(4)
        ]

    pending = {0: fire(0, 0)}
    for b in range(BPW):
        par = b % 2
        if b + 1 < BPW:
            pending[1 - par] = fire(b + 1, 1 - par)
        for cp in pending[par]:
            cp.wait()

        def red_body(rr, accs):
            a0, a1, a2, a3 = accs
            for u in range(4):
                r = rr * 4 + u
                a0 = a0 + rows_v[par, r, pl.ds(0, LANES)]
                a1 = a1 + rows_v[par, r, pl.ds(LANES, LANES)]
                a2 = a2 + rows_v[par, r, pl.ds(2 * LANES, LANES)]
                a3 = a3 + rows_v[par, r, pl.ds(3 * LANES, LANES)]
            return a0, a1, a2, a3

        z = jnp.zeros((LANES,), jnp.float32)
        acc = lax.fori_loop(0, IDX_PAD // 4, red_body, (z, z, z, z))
        for d in range(4):
            out_v[b, pl.ds(d * LANES, LANES)] = acc[d]

    pltpu.sync_copy(out_v, out_ref.at[pl.ds(wid * BPW, BPW)])


@jax.jit
def _run(sent_f, wn_f, wl2):
    mesh = plsc.VectorSubcoreMesh(core_axis_name="c", subcore_axis_name="s")
    return pl.kernel(
        _body,
        out_type=jax.ShapeDtypeStruct((B, D), jnp.float32),
        mesh=mesh,
        scratch_types=[
            pltpu.VMEM((TPW,), jnp.int32),
            pltpu.VMEM((NIDX // 128, 128), jnp.int32),
            pltpu.VMEM((NIDX // 128, 128), jnp.float32),
            pltpu.VMEM((BPW, 4, 128), jnp.int32),
            pltpu.VMEM((2, IDX_PAD, D), jnp.float32),
            pltpu.VMEM((BPW, D), jnp.float32),
            pltpu.SemaphoreType.DMA,
            pltpu.SemaphoreType.DMA((2,)),
        ],
        compiler_params=pltpu.CompilerParams(use_tc_tiling_on_sc=False,
                                             needs_layout_passes=False),
    )(sent_f, wn_f, wl2)


def kernel(sent, W_ngram_idx, W_lookup):
    sent_f = sent.astype(jnp.int32).reshape(-1)
    wn_lin = _relayout_wn(W_ngram_idx.T)
    wn_f = wn_lin.reshape(-1)
    wl_lin = _relayout_wl(W_lookup.T)
    wl2 = wl_lin.reshape(NWL_PAD, D)
    return _run(sent_f, wn_f, wl2)
